# Initial kernel scaffold; baseline (speedup 1.0000x reference)
import jax
import jax.numpy as jnp
from jax.experimental import pallas as pl


def kernel(x):
    raise NotImplementedError("write your pallas kernel here")


# speedup vs baseline: 11.8367x; 11.8367x over previous
import jax
import jax.numpy as jnp
from jax.experimental import pallas as pl
from jax.experimental.pallas import tpu as pltpu

_B, _C, _H, _W = 2, 5, 721, 1440
_K = 50
_DX = 25000.0
_DY = 25000.0
_VORT_THR = 1.4e-4
_FILL = -9999.0
_NEG = -3.0e38
_BIGF = 3.0e38
_HP = 728
_HPW = 736


def _tc_body(x_ref, out_ref, mslp_ref, w10p_ref, m_ref, rmax_ref):
    u10 = x_ref[0]
    v10 = x_ref[1]
    msl = x_ref[2]
    u850 = x_ref[3]
    v850 = x_ref[4]

    du = jnp.concatenate(
        [u850[1:2] - u850[0:1],
         (u850[2:] - u850[:-2]) / 2.0,
         u850[_H - 1:_H] - u850[_H - 2:_H - 1]], axis=0) / _DX
    dv = jnp.concatenate(
        [v850[:, 1:2] - v850[:, 0:1],
         (v850[:, 2:] - v850[:, :-2]) / 2.0,
         v850[:, _W - 1:_W] - v850[:, _W - 2:_W - 1]], axis=1) / _DY
    vort = du + dv

    up = jnp.concatenate([vort[1:], vort[:1]], axis=0)
    dn = jnp.concatenate([vort[_H - 1:], vort[:_H - 1]], axis=0)
    m1 = jnp.maximum(jnp.maximum(vort, up), dn)
    lf = jnp.concatenate([m1[:, 1:], m1[:, :1]], axis=1)
    rt = jnp.concatenate([m1[:, _W - 1:], m1[:, :_W - 1]], axis=1)
    m2 = jnp.maximum(jnp.maximum(m1, lf), rt)
    is_peak = (vort >= m2) & (vort > _VORT_THR)
    masked = jnp.concatenate(
        [jnp.where(is_peak, vort, _NEG),
         jnp.full((_HP - _H, _W), _NEG, jnp.float32)], axis=0)
    m_ref[:, :] = masked
    rmax_ref[:, :] = jnp.max(masked, axis=1, keepdims=True)

    w10 = jnp.sqrt(u10 * u10 + v10 * v10)
    pad = jnp.zeros((_HPW - _H - 4, _W), jnp.float32)
    mslp_ref[:, :] = jnp.concatenate(
        [msl[_H - 2:_H], msl, msl[0:2], pad], axis=0)
    w10p_ref[:, :] = jnp.concatenate(
        [w10[_H - 2:_H], w10, w10[0:2], pad], axis=0)

    iota_r = jax.lax.broadcasted_iota(jnp.int32, (_HP, 1), 0)
    iota_r8 = jax.lax.broadcasted_iota(jnp.int32, (8, 1), 0)
    iota_r16 = jax.lax.broadcasted_iota(jnp.int32, (16, 1), 0)
    iota_c8 = jax.lax.broadcasted_iota(jnp.int32, (8, _W), 1)
    iota_c16 = jax.lax.broadcasted_iota(jnp.int32, (16, _W), 1)
    c4 = jax.lax.broadcasted_iota(jnp.int32, (1, 4), 1)

    for k in range(_K):
        rmax = rmax_ref[:, :]
        rm = jnp.max(rmax)
        ri = jnp.min(jnp.where(rmax == rm, iota_r, _HP))
        base = pl.multiple_of((ri // 8) * 8, 8)
        off = ri - base
        blk = m_ref[pl.ds(base, 8), :]
        rowsel = iota_r8 == off
        vals = jnp.where(rowsel, blk, _NEG)
        cm = jnp.max(vals)
        ci = jnp.min(jnp.where(vals == cm, iota_c8, _W))
        newblk = jnp.where(rowsel & (iota_c8 == ci), _NEG, blk)
        m_ref[pl.ds(base, 8), :] = newblk
        rmax_ref[pl.ds(base, 8), :] = jnp.max(newblk, axis=1, keepdims=True)
        wsel = (iota_r16 >= off) & (iota_r16 <= off + 4)
        colmask = ((iota_c16 - ci + (2 + _W)) % _W) < 5
        wmask = wsel & colmask
        msl16 = mslp_ref[pl.ds(base, 16), :]
        w1016 = w10p_ref[pl.ds(base, 16), :]
        mslc = jnp.min(jnp.where(wmask, msl16, _BIGF))
        w10c = jnp.max(jnp.where(wmask, w1016, -_BIGF))
        valid = rm > _VORT_THR
        latv = jnp.where(valid, 90.0 - 0.25 * ri.astype(jnp.float32), _FILL)
        lonv = jnp.where(valid, 0.25 * ci.astype(jnp.float32), _FILL)
        mslv = jnp.where(valid, mslc, _FILL)
        w10v = jnp.where(valid, w10c, _FILL)
        vec = jnp.where(c4 == 0, latv,
                        jnp.where(c4 == 1, lonv,
                                  jnp.where(c4 == 2, mslv, w10v)))
        out_ref[k:k + 1, :] = vec


def _one_batch(xb):
    return pl.pallas_call(
        _tc_body,
        in_specs=[pl.BlockSpec((_C, _H, _W), lambda: (0, 0, 0))],
        out_specs=pl.BlockSpec((_K, 4), lambda: (0, 0)),
        out_shape=jax.ShapeDtypeStruct((_K, 4), jnp.float32),
        scratch_shapes=[
            pltpu.VMEM((_HPW, _W), jnp.float32),
            pltpu.VMEM((_HPW, _W), jnp.float32),
            pltpu.VMEM((_HP, _W), jnp.float32),
            pltpu.VMEM((_HP, 1), jnp.float32),
        ],
    )(xb)


def kernel(x):
    return jnp.stack([_one_batch(x[b]) for b in range(_B)])
